# baseline (speedup 1.0000x reference)
import functools

import jax
import jax.numpy as jnp
from jax import lax
from jax.experimental import pallas as pl
from jax.experimental.pallas import tpu as pltpu
from jax.experimental.pallas import tpu_sc as plsc


def _build_gather(B, At, Nbr, F, NC, NW, T):
    n_atoms = B * At
    mesh = plsc.VectorSubcoreMesh(core_axis_name="c", subcore_axis_name="s")

    @functools.partial(
        pl.kernel,
        mesh=mesh,
        out_type=jax.ShapeDtypeStruct((B, At, Nbr, Nbr, F), jnp.float32),
        compiler_params=pltpu.CompilerParams(use_tc_tiling_on_sc=False),
        scratch_types=[
            pltpu.VMEM((T, Nbr), jnp.int32),
            [pltpu.VMEM((Nbr, Nbr, F), jnp.float32) for _ in range(2)],
            [pltpu.SemaphoreType.DMA for _ in range(2)],
            [pltpu.SemaphoreType.DMA for _ in range(2)],
        ],
    )
    def k(table_hbm, idx_hbm, out_hbm, idx_v, bufs, gsems, ssems):
        wid = lax.axis_index("s") * NC + lax.axis_index("c")
        pltpu.sync_copy(idx_hbm.at[wid], idx_v)

        def dst(t):
            g = wid + NW * t
            b = g // At
            return out_hbm.at[b, g - b * At]

        def gather_start(t, bf):
            pltpu.async_copy(table_hbm.at[idx_v.at[t]], bufs[bf], gsems[bf])

        def gather_wait(t, bf):
            pltpu.make_async_copy(
                table_hbm.at[idx_v.at[t]], bufs[bf], gsems[bf]).wait()

        def scatter_start(t, bf):
            pltpu.async_copy(bufs[bf], dst(t), ssems[bf])

        def scatter_wait(t, bf):
            pltpu.make_async_copy(bufs[bf], dst(t), ssems[bf]).wait()

        def visit(t, bf, first=False, fire_next=True):
            gather_wait(t, bf)
            scatter_start(t, bf)
            if not first:
                scatter_wait(t - 1, 1 - bf)
            if fire_next:
                gather_start(t + 1, 1 - bf)

        gather_start(0, 0)
        visit(0, 0, first=True)

        @pl.loop(0, (T - 3) // 2)
        def _body(i):
            t = 1 + 2 * i
            visit(t, 1)
            visit(t + 1, 0)

        visit(T - 2, 1)

        gather_wait(T - 1, 0)
        last_valid = wid + NW * (T - 1) < n_atoms

        @pl.when(last_valid)
        def _last_scatter():
            scatter_start(T - 1, 0)

        scatter_wait(T - 2, 1)

        @pl.when(last_valid)
        def _last_drain():
            scatter_wait(T - 1, 0)

    return k


def kernel(edge_embedding, nbr_idx):
    B, At, Nbr, F = edge_embedding.shape
    n_atoms = B * At

    info = plsc.get_sparse_core_info()
    NC, NS = info.num_cores, info.num_subcores
    NW = NC * NS
    T = -(-n_atoms // NW)

    table = edge_embedding.reshape(n_atoms, Nbr, F)
    idx = nbr_idx.astype(jnp.int32).reshape(B, At * Nbr)
    idx = idx + (jnp.arange(B, dtype=jnp.int32) * At)[:, None]
    idx = idx.reshape(n_atoms, Nbr)
    idx = jnp.pad(idx, ((0, T * NW - n_atoms), (0, 0)))
    idx = idx.reshape(T, NW, Nbr).transpose(1, 0, 2)

    return _build_gather(B, At, Nbr, F, NC, NW, T)(table, idx)


# speedup vs baseline: 4.0009x; 4.0009x over previous
import functools

import jax
import jax.numpy as jnp
from jax import lax
from jax.experimental import pallas as pl
from jax.experimental.pallas import tpu as pltpu
from jax.experimental.pallas import tpu_sc as plsc

_BAND = 16
_L = 16


def _build(B, At, Nbr, F, NC, NW):
    C = Nbr * F
    NBANDS = C // _BAND
    P = (B * Nbr) // NW
    NCH = -(-At // _L)
    At_pad = NCH * _L
    mesh = plsc.VectorSubcoreMesh(core_axis_name="c", subcore_axis_name="s")

    @functools.partial(
        pl.kernel,
        mesh=mesh,
        out_type=jax.ShapeDtypeStruct((B, Nbr, C, At), jnp.float32),
        compiler_params=pltpu.CompilerParams(needs_layout_passes=False),
        scratch_types=[
            pltpu.VMEM((At_pad,), jnp.int32),
            [pltpu.VMEM((_BAND, At), jnp.float32) for _ in range(2)],
            [pltpu.VMEM((_BAND, At), jnp.float32) for _ in range(2)],
            [pltpu.SemaphoreType.DMA for _ in range(2)],
            [pltpu.SemaphoreType.DMA for _ in range(2)],
        ],
    )
    def k(table_hbm, idx_hbm, out_hbm, idx_v, ibufs, obufs, isems, osems):
        wid = lax.axis_index("s") * NC + lax.axis_index("c")

        for p in range(P):
            pid = wid * P + p
            b = pid // Nbr
            n1 = pid - b * Nbr
            pltpu.sync_copy(idx_hbm.at[b, n1], idx_v)

            def in_start(t, bf, b=b):
                pltpu.async_copy(
                    table_hbm.at[b, pl.ds(t * _BAND, _BAND)],
                    ibufs[bf], isems[bf])

            def in_wait(t, bf, b=b):
                pltpu.make_async_copy(
                    table_hbm.at[b, pl.ds(t * _BAND, _BAND)],
                    ibufs[bf], isems[bf]).wait()

            def out_start(t, bf, b=b, n1=n1):
                pltpu.async_copy(
                    obufs[bf],
                    out_hbm.at[b, n1, pl.ds(t * _BAND, _BAND)], osems[bf])

            def out_wait(t, bf, b=b, n1=n1):
                pltpu.make_async_copy(
                    obufs[bf],
                    out_hbm.at[b, n1, pl.ds(t * _BAND, _BAND)],
                    osems[bf]).wait()

            def compute(bf):
                @pl.loop(0, NCH)
                def _chunks(j):
                    a0 = lax.min(j * _L, At - _L)
                    iv = idx_v[pl.ds(a0, _L)]
                    for r in range(_BAND):
                        row = jnp.full((_L,), r, jnp.int32)
                        vals = plsc.load_gather(ibufs[bf], [row, iv])
                        obufs[bf][r, pl.ds(a0, _L)] = vals

            def visit(t, bf, fire_in=True, wait_out=True):
                in_wait(t, bf)
                if fire_in:
                    in_start(t + 1, 1 - bf)
                if wait_out:
                    out_wait(t - 2, bf)
                compute(bf)
                out_start(t, bf)

            in_start(0, 0)
            visit(0, 0, wait_out=False)
            visit(1, 1, wait_out=False)

            @pl.loop(0, (NBANDS - 4) // 2)
            def _body(i):
                t = 2 + 2 * i
                visit(t, 0)
                visit(t + 1, 1)

            visit(NBANDS - 2, 0)
            visit(NBANDS - 1, 1, fire_in=False)
            out_wait(NBANDS - 2, 0)
            out_wait(NBANDS - 1, 1)

    return k


def kernel(edge_embedding, nbr_idx):
    B, At, Nbr, F = edge_embedding.shape
    C = Nbr * F

    info = plsc.get_sparse_core_info()
    NC, NS = info.num_cores, info.num_subcores
    NW = NC * NS

    table = edge_embedding.reshape(B, At, C).transpose(0, 2, 1)
    idxT = nbr_idx.astype(jnp.int32).transpose(0, 2, 1)
    pad = (-At) % _L
    idxT = jnp.pad(idxT, ((0, 0), (0, 0), (0, pad)))

    x3 = _build(B, At, Nbr, F, NC, NW)(table, idxT)
    return x3.reshape(B, Nbr, Nbr, F, At).transpose(0, 4, 1, 2, 3)


# speedup vs baseline: 7.2882x; 1.8216x over previous
import functools

import jax
import jax.numpy as jnp
from jax import lax
from jax.experimental import pallas as pl
from jax.experimental.pallas import tpu as pltpu
from jax.experimental.pallas import tpu_sc as plsc

_BAND = 16
_L = 16


def _build(B, At, Nbr, F, NC, NW):
    C = Nbr * F
    NBANDS = C // _BAND
    P = (B * Nbr) // NW
    NCH = -(-At // _L)
    At_pad = NCH * _L
    mesh = plsc.VectorSubcoreMesh(core_axis_name="c", subcore_axis_name="s")

    @functools.partial(
        pl.kernel,
        mesh=mesh,
        out_type=jax.ShapeDtypeStruct((B, Nbr, C, At), jnp.float32),
        compiler_params=pltpu.CompilerParams(needs_layout_passes=False),
        scratch_types=[
            pltpu.VMEM((At_pad,), jnp.int32),
            [pltpu.VMEM((_BAND, At), jnp.float32) for _ in range(2)],
            [pltpu.VMEM((_BAND, At), jnp.float32) for _ in range(2)],
            [pltpu.SemaphoreType.DMA for _ in range(2)],
            [pltpu.SemaphoreType.DMA for _ in range(2)],
        ],
    )
    def k(table_hbm, idx_hbm, out_hbm, idx_v, ibufs, obufs, isems, osems):
        wid = lax.axis_index("s") * NC + lax.axis_index("c")

        for p in range(P):
            pid = wid * P + p
            b = pid // Nbr
            n1 = pid - b * Nbr
            pltpu.sync_copy(idx_hbm.at[b, n1], idx_v)

            def in_start(t, bf, b=b):
                pltpu.async_copy(
                    table_hbm.at[b, pl.ds(t * _BAND, _BAND)],
                    ibufs[bf], isems[bf])

            def in_wait(t, bf, b=b):
                pltpu.make_async_copy(
                    table_hbm.at[b, pl.ds(t * _BAND, _BAND)],
                    ibufs[bf], isems[bf]).wait()

            def out_start(t, bf, b=b, n1=n1):
                pltpu.async_copy(
                    obufs[bf],
                    out_hbm.at[b, n1, pl.ds(t * _BAND, _BAND)], osems[bf])

            def out_wait(t, bf, b=b, n1=n1):
                pltpu.make_async_copy(
                    obufs[bf],
                    out_hbm.at[b, n1, pl.ds(t * _BAND, _BAND)],
                    osems[bf]).wait()

            def compute(bf):
                @pl.loop(0, NCH)
                def _chunks(j):
                    a0 = lax.min(j * _L, At - _L)
                    iv = idx_v[pl.ds(a0, _L)]
                    vals = []
                    for r in range(_BAND):
                        row = jnp.full((_L,), r, jnp.int32)
                        vals.append(plsc.load_gather(ibufs[bf], [row, iv]))
                    for r in range(_BAND):
                        obufs[bf][r, pl.ds(a0, _L)] = vals[r]

            def visit(t, bf, fire_in=True, wait_out=True):
                in_wait(t, bf)
                if fire_in:
                    in_start(t + 1, 1 - bf)
                if wait_out:
                    out_wait(t - 2, bf)
                compute(bf)
                out_start(t, bf)

            in_start(0, 0)
            visit(0, 0, wait_out=False)
            visit(1, 1, wait_out=False)

            @pl.loop(0, (NBANDS - 4) // 2)
            def _body(i):
                t = 2 + 2 * i
                visit(t, 0)
                visit(t + 1, 1)

            visit(NBANDS - 2, 0)
            visit(NBANDS - 1, 1, fire_in=False)
            out_wait(NBANDS - 2, 0)
            out_wait(NBANDS - 1, 1)

    return k


def kernel(edge_embedding, nbr_idx):
    B, At, Nbr, F = edge_embedding.shape
    C = Nbr * F

    info = plsc.get_sparse_core_info()
    NC, NS = info.num_cores, info.num_subcores
    NW = NC * NS

    table = edge_embedding.reshape(B, At, C).transpose(0, 2, 1)
    idxT = nbr_idx.astype(jnp.int32).transpose(0, 2, 1)
    pad = (-At) % _L
    idxT = jnp.pad(idxT, ((0, 0), (0, 0), (0, pad)))

    x3 = _build(B, At, Nbr, F, NC, NW)(table, idxT)
    return x3.reshape(B, Nbr, Nbr, F, At).transpose(0, 4, 1, 2, 3)
